# baseline (speedup 1.0000x reference)
import functools

import jax
import jax.numpy as jnp
from jax import lax
from jax.experimental import pallas as pl
from jax.experimental.pallas import tpu as pltpu
from jax.experimental.pallas import tpu_sc as plsc

_NC = 2
_NS = 16
_NW = _NC * _NS

_CHUNK = 128
_N_CHUNKS = 126
_PER_W = _CHUNK * _N_CHUNKS
_B_PAD = _NW * _PER_W

_D = 64
_ROWS_TC = 4000


def _softplus(x):
  return jnp.maximum(x, 0.0) + jnp.log1p(jnp.exp(-jnp.abs(x)))


def _atom_tables_body(a_ref, w1a_ref, w1b_ref, o1_ref, o2_ref):
  a = a_ref[...]
  o1_ref[...] = jnp.dot(
      a, w1a_ref[...], preferred_element_type=jnp.float32
  ).astype(jnp.bfloat16)
  o2_ref[...] = jnp.dot(
      a, w1b_ref[...], preferred_element_type=jnp.float32
  ).astype(jnp.bfloat16)


def _glob_table_body(g_ref, wg_ref, b1_ref, o_ref):
  o_ref[...] = (
      jnp.dot(g_ref[...], wg_ref[...], preferred_element_type=jnp.float32)
      + b1_ref[...]
  ).astype(jnp.bfloat16)


_IDX_PER_W = _N_CHUNKS * _CHUNK


def _sc_gather_body(a1_hbm, a2_hbm, g1_hbm, i0_hbm, i1_hbm, ig_hbm, out_hbm,
                    idx0_v, idx1_v, idxg_v,
                    bufa0, bufb0, bufg0, outb0,
                    bufa1, bufb1, bufg1, outb1,
                    bufa2, bufb2, bufg2, outb2,
                    sem_g0, sem_g1, sem_g2, sem_o0, sem_o1, sem_o2):
  wid = lax.axis_index("s") * _NC + lax.axis_index("c")
  base = wid * _PER_W
  pbase0 = wid * (_PER_W // 2)

  pltpu.sync_copy(i0_hbm.at[pl.ds(base, _PER_W)], idx0_v)
  pltpu.sync_copy(i1_hbm.at[pl.ds(base, _PER_W)], idx1_v)
  pltpu.sync_copy(ig_hbm.at[pl.ds(base, _PER_W)], idxg_v)

  sets = (
      (bufa0, bufb0, bufg0, outb0, sem_g0, sem_o0),
      (bufa1, bufb1, bufg1, outb1, sem_g1, sem_o1),
      (bufa2, bufb2, bufg2, outb2, sem_g2, sem_o2),
  )

  def issue(s, j):
    ba, bb, bg, _, sg, _ = sets[s]
    off = j * _CHUNK
    pltpu.async_copy(a1_hbm.at[idx0_v.at[pl.ds(off, _CHUNK)]], ba, sg)
    pltpu.async_copy(a2_hbm.at[idx1_v.at[pl.ds(off, _CHUNK)]], bb, sg)
    pltpu.async_copy(g1_hbm.at[idxg_v.at[pl.ds(off, _CHUNK)]], bg, sg)

  def drain_gathers(s):
    ba, bb, bg, _, sg, _ = sets[s]
    pltpu.make_async_copy(a1_hbm.at[idx0_v.at[pl.ds(0, _CHUNK)]], ba, sg).wait()
    pltpu.make_async_copy(a2_hbm.at[idx1_v.at[pl.ds(0, _CHUNK)]], bb, sg).wait()
    pltpu.make_async_copy(g1_hbm.at[idxg_v.at[pl.ds(0, _CHUNK)]], bg, sg).wait()

  def drain_out(s):
    _, _, _, ob, _, so = sets[s]
    pltpu.make_async_copy(ob, out_hbm.at[pl.ds(0, _CHUNK // 2)], so).wait()

  def consume(s, j, first_round):
    ba, bb, bg, ob, sg, so = sets[s]
    drain_gathers(s)

    @pl.when(jnp.logical_not(first_round))
    def _():
      drain_out(s)

    def row_body(r, c2):
      for h in range(2):
        b = 2 * r + h
        for k in range(_D // 32):
          sl = pl.ds(k * 32, 32)
          ssum = ba[b, sl] + bb[b, sl] + bg[b, sl]
          lo, hi = plsc.unpack(ssum, format=plsc.PackFormat.INTERLEAVED)
          ob[r, pl.ds(h * _D + k * 32, 16)] = lo
          ob[r, pl.ds(h * _D + k * 32 + 16, 16)] = hi
      return c2

    lax.fori_loop(0, _CHUNK // 2, row_body, 0, unroll=4)
    pltpu.async_copy(
        ob, out_hbm.at[pl.ds(pbase0 + j * (_CHUNK // 2), _CHUNK // 2)], so)

  issue(0, 0)
  issue(1, 1)

  def tri_body(t, carry):
    for s in range(3):
      j = 3 * t + s

      @pl.when(j + 2 < _N_CHUNKS)
      def _():
        issue((s + 2) % 3, j + 2)

      consume(s, j, t == 0)
    return carry

  lax.fori_loop(0, _N_CHUNKS // 3, tri_body, 0)
  drain_out(0)
  drain_out(1)
  drain_out(2)


def _mlp_body(x_ref, s_ref, w1m_ref, w2_ref, b2_ref, w3_ref, b3_ref, o_ref):
  h = jnp.dot(x_ref[...], w1m_ref[...], preferred_element_type=jnp.float32)
  h = _softplus(h + s_ref[...])
  h = _softplus(
      jnp.dot(h, w2_ref[...], preferred_element_type=jnp.float32) + b2_ref[...]
  )
  o_ref[...] = (
      jnp.dot(h, w3_ref[...], preferred_element_type=jnp.float32) + b3_ref[...]
  )


def kernel(master_feats, atom_feats, global_feats, a2b_src, g2b_src,
           W1, b1, W2, b2, W3, b3):
  n_bonds = master_feats.shape[0]
  n_atoms = atom_feats.shape[0]
  d = _D

  W1m = W1[:d]
  j16 = jnp.arange(16)
  grp = jnp.stack([j16, j16 + 16], axis=1).reshape(32)
  perm = jnp.concatenate([grp, grp + 32])
  W1a = W1[d:2 * d][:, perm]
  W1b = W1[2 * d:3 * d][:, perm]
  W1g = W1[3 * d:4 * d][:, perm]
  b1p = b1[perm]

  rows1 = 10000
  grid1 = n_atoms // rows1
  A1, A2 = pl.pallas_call(
      _atom_tables_body,
      grid=(grid1,),
      in_specs=[
          pl.BlockSpec((rows1, d), lambda i: (i, 0)),
          pl.BlockSpec((d, d), lambda i: (0, 0)),
          pl.BlockSpec((d, d), lambda i: (0, 0)),
      ],
      out_specs=[
          pl.BlockSpec((rows1, d), lambda i: (i, 0)),
          pl.BlockSpec((rows1, d), lambda i: (i, 0)),
      ],
      out_shape=[
          jax.ShapeDtypeStruct((n_atoms, d), jnp.bfloat16),
          jax.ShapeDtypeStruct((n_atoms, d), jnp.bfloat16),
      ],
  )(atom_feats, W1a, W1b)

  G1 = pl.pallas_call(
      _glob_table_body,
      out_shape=jax.ShapeDtypeStruct((global_feats.shape[0], d), jnp.bfloat16),
  )(global_feats, W1g, b1p.reshape(1, d))

  pad = _B_PAD - n_bonds
  i0 = jnp.pad(a2b_src[:, 0], (0, pad))
  i1 = jnp.pad(a2b_src[:, 1], (0, pad))
  ig = jnp.pad(g2b_src, (0, pad))

  mesh = plsc.VectorSubcoreMesh(
      core_axis_name="c", subcore_axis_name="s",
      num_cores=_NC, num_subcores=_NS,
  )
  buf_set = [
      pltpu.VMEM((_CHUNK, d), jnp.bfloat16),
      pltpu.VMEM((_CHUNK, d), jnp.bfloat16),
      pltpu.VMEM((_CHUNK, d), jnp.bfloat16),
      pltpu.VMEM((_CHUNK // 2, 2 * d), jnp.float32),
  ]
  sc_gather = pl.kernel(
      _sc_gather_body,
      out_type=jax.ShapeDtypeStruct((_B_PAD // 2, 2 * d), jnp.float32),
      mesh=mesh,
      compiler_params=pltpu.CompilerParams(
          use_tc_tiling_on_sc=False, needs_layout_passes=False),
      scratch_types=(
          [pltpu.VMEM((_PER_W,), jnp.int32)] * 3
          + buf_set * 3
          + [pltpu.SemaphoreType.DMA] * 6
      ),
  )
  S2 = sc_gather(A1, A2, G1, i0, i1, ig)

  eye2 = jnp.eye(2, dtype=jnp.float32)
  W1bd = jnp.kron(eye2, W1m)
  W2bd = jnp.kron(eye2, W2)
  W3bd = jnp.kron(eye2, W3)
  b2bd = jnp.tile(b2, 2).reshape(1, 2 * d)
  b3bd = jnp.tile(b3, 2).reshape(1, 64)
  master2 = master_feats.reshape(n_bonds // 2, 2 * d)

  rows3 = _ROWS_TC // 2
  grid3 = (n_bonds // 2) // rows3
  out = pl.pallas_call(
      _mlp_body,
      grid=(grid3,),
      in_specs=[
          pl.BlockSpec((rows3, 2 * d), lambda i: (i, 0)),
          pl.BlockSpec((rows3, 2 * d), lambda i: (i, 0)),
          pl.BlockSpec((2 * d, 2 * d), lambda i: (0, 0)),
          pl.BlockSpec((2 * d, 2 * d), lambda i: (0, 0)),
          pl.BlockSpec((1, 2 * d), lambda i: (0, 0)),
          pl.BlockSpec((2 * d, 64), lambda i: (0, 0)),
          pl.BlockSpec((1, 64), lambda i: (0, 0)),
      ],
      out_specs=pl.BlockSpec((rows3, 64), lambda i: (i, 0)),
      out_shape=jax.ShapeDtypeStruct((n_bonds // 2, 64), jnp.float32),
  )(master2, S2, W1bd, W2bd, b2bd, W3bd, b3bd)

  return out.reshape(n_bonds, 32)


# speedup vs baseline: 1.0635x; 1.0635x over previous
import functools

import jax
import jax.numpy as jnp
from jax import lax
from jax.experimental import pallas as pl
from jax.experimental.pallas import tpu as pltpu
from jax.experimental.pallas import tpu_sc as plsc

_NC = 2
_NS = 16
_NW = _NC * _NS

_CHUNK = 128
_N_CHUNKS = 124
_PER_W = _CHUNK * _N_CHUNKS
_B_PAD = _NW * _PER_W

_D = 64
_ROWS_TC = 4000


def _softplus(x):
  return jnp.maximum(x, 0.0) + jnp.log1p(jnp.exp(-jnp.abs(x)))


def _atom_tables_body(a_ref, w1a_ref, w1b_ref, o1_ref, o2_ref):
  a = a_ref[...]
  o1_ref[...] = jnp.dot(
      a, w1a_ref[...], preferred_element_type=jnp.float32
  ).astype(jnp.bfloat16)
  o2_ref[...] = jnp.dot(
      a, w1b_ref[...], preferred_element_type=jnp.float32
  ).astype(jnp.bfloat16)


def _glob_table_body(g_ref, wg_ref, b1_ref, o_ref):
  o_ref[...] = (
      jnp.dot(g_ref[...], wg_ref[...], preferred_element_type=jnp.float32)
      + b1_ref[...]
  ).astype(jnp.bfloat16)


_IDX_PER_W = _N_CHUNKS * 3 * _CHUNK


def _sc_gather_body(a1_hbm, a2_hbm, g1_hbm, idx_hbm, out_hbm,
                    idx_v, bufa0, bufb0, bufg0, bufa1, bufb1, bufg1,
                    outb0, outb1, sem_g0, sem_g1, sem_o0, sem_o1):
  wid = lax.axis_index("s") * _NC + lax.axis_index("c")
  pbase0 = wid * (_PER_W // 2)

  pltpu.sync_copy(idx_hbm.at[wid], idx_v)

  sets = (
      (bufa0, bufb0, bufg0, outb0, sem_g0, sem_o0),
      (bufa1, bufb1, bufg1, outb1, sem_g1, sem_o1),
  )

  def issue(s, j):
    ba, bb, bg, _, sg, _ = sets[s]
    off = j * (3 * _CHUNK)
    pltpu.async_copy(a1_hbm.at[idx_v.at[pl.ds(off, _CHUNK)]], ba, sg)
    pltpu.async_copy(a2_hbm.at[idx_v.at[pl.ds(off + _CHUNK, _CHUNK)]], bb, sg)
    pltpu.async_copy(g1_hbm.at[idx_v.at[pl.ds(off + 2 * _CHUNK, _CHUNK)]],
                     bg, sg)

  def drain_gathers(s):
    ba, bb, bg, _, sg, _ = sets[s]
    pltpu.make_async_copy(a1_hbm.at[idx_v.at[pl.ds(0, _CHUNK)]], ba, sg).wait()
    pltpu.make_async_copy(a2_hbm.at[idx_v.at[pl.ds(0, _CHUNK)]], bb, sg).wait()
    pltpu.make_async_copy(g1_hbm.at[idx_v.at[pl.ds(0, _CHUNK)]], bg, sg).wait()

  def drain_out(s):
    _, _, _, ob, _, so = sets[s]
    pltpu.make_async_copy(ob, out_hbm.at[pl.ds(0, _CHUNK // 2)], so).wait()

  def consume(s, j):
    ba, bb, bg, ob, sg, so = sets[s]
    drain_gathers(s)

    def row_body(r, c2):
      for h in range(2):
        b = 2 * r + h
        for k in range(_D // 32):
          sl = pl.ds(k * 32, 32)
          ssum = ba[b, sl] + bb[b, sl] + bg[b, sl]
          lo, hi = plsc.unpack(ssum, format=plsc.PackFormat.INTERLEAVED)
          ob[r, pl.ds(h * _D + k * 32, 16)] = lo
          ob[r, pl.ds(h * _D + k * 32 + 16, 16)] = hi
      return c2

    lax.fori_loop(0, _CHUNK // 2, row_body, 0, unroll=4)
    pltpu.async_copy(
        ob, out_hbm.at[pl.ds(pbase0 + j * (_CHUNK // 2), _CHUNK // 2)], so)

  issue(0, 0)

  def pair_body(t, carry):
    j1 = 2 * t + 1
    j0n = 2 * t + 2

    @pl.when(t > 0)
    def _():
      drain_out(1)
    issue(1, j1)

    consume(0, 2 * t)

    @pl.when(j0n < _N_CHUNKS)
    def _():
      drain_out(0)
      issue(0, j0n)

    consume(1, j1)
    return carry

  lax.fori_loop(0, _N_CHUNKS // 2, pair_body, 0)
  drain_out(0)
  drain_out(1)


def _mlp_body(x_ref, s_ref, w1m_ref, w2_ref, b2_ref, w3_ref, b3_ref, o_ref):
  h = jnp.dot(x_ref[...], w1m_ref[...], preferred_element_type=jnp.float32)
  h = _softplus(h + s_ref[...])
  h = _softplus(
      jnp.dot(h, w2_ref[...], preferred_element_type=jnp.float32) + b2_ref[...]
  )
  o_ref[...] = (
      jnp.dot(h, w3_ref[...], preferred_element_type=jnp.float32) + b3_ref[...]
  )


def kernel(master_feats, atom_feats, global_feats, a2b_src, g2b_src,
           W1, b1, W2, b2, W3, b3):
  n_bonds = master_feats.shape[0]
  n_atoms = atom_feats.shape[0]
  d = _D

  W1m = W1[:d]
  j16 = jnp.arange(16)
  grp = jnp.stack([j16, j16 + 16], axis=1).reshape(32)
  perm = jnp.concatenate([grp, grp + 32])
  W1a = W1[d:2 * d][:, perm]
  W1b = W1[2 * d:3 * d][:, perm]
  W1g = W1[3 * d:4 * d][:, perm]
  b1p = b1[perm]

  rows1 = 10000
  grid1 = n_atoms // rows1
  A1, A2 = pl.pallas_call(
      _atom_tables_body,
      grid=(grid1,),
      in_specs=[
          pl.BlockSpec((rows1, d), lambda i: (i, 0)),
          pl.BlockSpec((d, d), lambda i: (0, 0)),
          pl.BlockSpec((d, d), lambda i: (0, 0)),
      ],
      out_specs=[
          pl.BlockSpec((rows1, d), lambda i: (i, 0)),
          pl.BlockSpec((rows1, d), lambda i: (i, 0)),
      ],
      out_shape=[
          jax.ShapeDtypeStruct((n_atoms, d), jnp.bfloat16),
          jax.ShapeDtypeStruct((n_atoms, d), jnp.bfloat16),
      ],
  )(atom_feats, W1a, W1b)

  G1 = pl.pallas_call(
      _glob_table_body,
      out_shape=jax.ShapeDtypeStruct((global_feats.shape[0], d), jnp.bfloat16),
  )(global_feats, W1g, b1p.reshape(1, d))

  pad = _B_PAD - n_bonds
  i0 = jnp.pad(a2b_src[:, 0], (0, pad)).reshape(_NW, _N_CHUNKS, 1, _CHUNK)
  i1 = jnp.pad(a2b_src[:, 1], (0, pad)).reshape(_NW, _N_CHUNKS, 1, _CHUNK)
  ig = jnp.pad(g2b_src, (0, pad)).reshape(_NW, _N_CHUNKS, 1, _CHUNK)
  idx_all = jnp.concatenate([i0, i1, ig], axis=2).reshape(_NW, _IDX_PER_W)

  mesh = plsc.VectorSubcoreMesh(
      core_axis_name="c", subcore_axis_name="s",
      num_cores=_NC, num_subcores=_NS,
  )
  sc_gather = pl.kernel(
      _sc_gather_body,
      out_type=jax.ShapeDtypeStruct((_B_PAD // 2, 2 * d), jnp.float32),
      mesh=mesh,
      compiler_params=pltpu.CompilerParams(
          use_tc_tiling_on_sc=False, needs_layout_passes=False),
      scratch_types=[
          pltpu.VMEM((_IDX_PER_W,), jnp.int32),
          pltpu.VMEM((_CHUNK, d), jnp.bfloat16),
          pltpu.VMEM((_CHUNK, d), jnp.bfloat16),
          pltpu.VMEM((_CHUNK, d), jnp.bfloat16),
          pltpu.VMEM((_CHUNK, d), jnp.bfloat16),
          pltpu.VMEM((_CHUNK, d), jnp.bfloat16),
          pltpu.VMEM((_CHUNK, d), jnp.bfloat16),
          pltpu.VMEM((_CHUNK // 2, 2 * d), jnp.float32),
          pltpu.VMEM((_CHUNK // 2, 2 * d), jnp.float32),
          pltpu.SemaphoreType.DMA,
          pltpu.SemaphoreType.DMA,
          pltpu.SemaphoreType.DMA,
          pltpu.SemaphoreType.DMA,
      ],
  )
  S2 = sc_gather(A1, A2, G1, idx_all)

  eye2 = jnp.eye(2, dtype=jnp.float32)
  W1bd = jnp.kron(eye2, W1m)
  W2bd = jnp.kron(eye2, W2)
  W3bd = jnp.kron(eye2, W3)
  b2bd = jnp.tile(b2, 2).reshape(1, 2 * d)
  b3bd = jnp.tile(b3, 2).reshape(1, 64)
  master2 = master_feats.reshape(n_bonds // 2, 2 * d)

  rows3 = _ROWS_TC // 2
  grid3 = (n_bonds // 2) // rows3
  out = pl.pallas_call(
      _mlp_body,
      grid=(grid3,),
      in_specs=[
          pl.BlockSpec((rows3, 2 * d), lambda i: (i, 0)),
          pl.BlockSpec((rows3, 2 * d), lambda i: (i, 0)),
          pl.BlockSpec((2 * d, 2 * d), lambda i: (0, 0)),
          pl.BlockSpec((2 * d, 2 * d), lambda i: (0, 0)),
          pl.BlockSpec((1, 2 * d), lambda i: (0, 0)),
          pl.BlockSpec((2 * d, 64), lambda i: (0, 0)),
          pl.BlockSpec((1, 64), lambda i: (0, 0)),
      ],
      out_specs=pl.BlockSpec((rows3, 64), lambda i: (i, 0)),
      out_shape=jax.ShapeDtypeStruct((n_bonds // 2, 64), jnp.float32),
  )(master2, S2, W1bd, W2bd, b2bd, W3bd, b3bd)

  return out.reshape(n_bonds, 32)


# speedup vs baseline: 1.1668x; 1.0972x over previous
import functools

import jax
import jax.numpy as jnp
from jax import lax
from jax.experimental import pallas as pl
from jax.experimental.pallas import tpu as pltpu
from jax.experimental.pallas import tpu_sc as plsc

_NC = 2
_NS = 16
_NW = _NC * _NS

_CHUNK = 128
_N_CHUNKS = 124
_PER_W = _CHUNK * _N_CHUNKS
_B_PAD = _NW * _PER_W

_D = 64
_ROWS_TC = 4000


def _softplus(x):
  return jnp.maximum(x, 0.0) + jnp.log1p(jnp.exp(-jnp.abs(x)))


def _atom_tables_body(a_ref, w1a_ref, w1b_ref, o1_ref, o2_ref):
  a = a_ref[...]
  o1_ref[...] = jnp.dot(
      a, w1a_ref[...], preferred_element_type=jnp.float32
  ).astype(jnp.bfloat16)
  o2_ref[...] = jnp.dot(
      a, w1b_ref[...], preferred_element_type=jnp.float32
  ).astype(jnp.bfloat16)


def _glob_table_body(g_ref, wg_ref, b1_ref, o_ref):
  o_ref[...] = (
      jnp.dot(g_ref[...], wg_ref[...], preferred_element_type=jnp.float32)
      + b1_ref[...]
  ).astype(jnp.bfloat16)


_IDX_PER_W = _N_CHUNKS * 3 * _CHUNK


def _sc_gather_body(a1_hbm, a2_hbm, g1_hbm, idx_hbm, out_hbm,
                    idx_v, bufa0, bufb0, bufg0, bufa1, bufb1, bufg1,
                    outb0, outb1, sem_g0, sem_g1, sem_o0, sem_o1):
  wid = lax.axis_index("s") * _NC + lax.axis_index("c")
  pbase0 = wid * (_PER_W // 2)

  pltpu.sync_copy(idx_hbm.at[wid], idx_v)

  sets = (
      (bufa0, bufb0, bufg0, outb0, sem_g0, sem_o0),
      (bufa1, bufb1, bufg1, outb1, sem_g1, sem_o1),
  )

  def issue(s, j):
    ba, bb, bg, _, sg, _ = sets[s]
    off = j * (3 * _CHUNK)
    pltpu.async_copy(a1_hbm.at[idx_v.at[pl.ds(off, _CHUNK)]], ba, sg)
    pltpu.async_copy(a2_hbm.at[idx_v.at[pl.ds(off + _CHUNK, _CHUNK)]], bb, sg)
    pltpu.async_copy(g1_hbm.at[idx_v.at[pl.ds(off + 2 * _CHUNK, _CHUNK)]],
                     bg, sg)

  def drain_gathers(s):
    ba, bb, bg, _, sg, _ = sets[s]
    pltpu.make_async_copy(a1_hbm.at[idx_v.at[pl.ds(0, _CHUNK)]], ba, sg).wait()
    pltpu.make_async_copy(a2_hbm.at[idx_v.at[pl.ds(0, _CHUNK)]], bb, sg).wait()
    pltpu.make_async_copy(g1_hbm.at[idx_v.at[pl.ds(0, _CHUNK)]], bg, sg).wait()

  def drain_out(s):
    _, _, _, ob, _, so = sets[s]
    pltpu.make_async_copy(ob, out_hbm.at[pl.ds(0, _CHUNK // 2)], so).wait()

  def consume(s, j):
    ba, bb, bg, ob, sg, so = sets[s]
    drain_gathers(s)

    def row_body(r, c2):
      for h in range(2):
        b = 2 * r + h
        for k in range(_D // 32):
          sl = pl.ds(k * 32, 32)
          ssum = ba[b, sl] + bb[b, sl] + bg[b, sl]
          lo, hi = plsc.unpack(ssum, format=plsc.PackFormat.INTERLEAVED)
          ob[r, pl.ds(h * _D + k * 32, 16)] = lo
          ob[r, pl.ds(h * _D + k * 32 + 16, 16)] = hi
      return c2

    lax.fori_loop(0, _CHUNK // 2, row_body, 0, unroll=4)
    pltpu.async_copy(
        ob, out_hbm.at[pl.ds(pbase0 + j * (_CHUNK // 2), _CHUNK // 2)], so)

  issue(0, 0)

  def pair_body(t, carry):
    j1 = 2 * t + 1
    j0n = 2 * t + 2

    @pl.when(t > 0)
    def _():
      drain_out(1)
    issue(1, j1)

    consume(0, 2 * t)

    @pl.when(j0n < _N_CHUNKS)
    def _():
      drain_out(0)
      issue(0, j0n)

    consume(1, j1)
    return carry

  lax.fori_loop(0, _N_CHUNKS // 2, pair_body, 0)
  drain_out(0)
  drain_out(1)


def _mlp_body(x_ref, s_ref, w1m_ref, w2_ref, b2_ref, w3_ref, b3_ref,
              oe_ref, oo_ref):
  h = jnp.dot(x_ref[...], w1m_ref[...], preferred_element_type=jnp.float32)
  h = _softplus(h + s_ref[...])
  h = _softplus(
      jnp.dot(h, w2_ref[...], preferred_element_type=jnp.float32) + b2_ref[...]
  )
  tpacked = lax.dot_general(
      w3_ref[...], h, (((0,), (1,)), ((), ())),
      preferred_element_type=jnp.float32,
  ) + b3_ref[...]
  oe_ref[...] = tpacked[:32, :]
  oo_ref[...] = tpacked[32:, :]


def kernel(master_feats, atom_feats, global_feats, a2b_src, g2b_src,
           W1, b1, W2, b2, W3, b3):
  n_bonds = master_feats.shape[0]
  n_atoms = atom_feats.shape[0]
  d = _D

  W1m = W1[:d]
  j16 = jnp.arange(16)
  grp = jnp.stack([j16, j16 + 16], axis=1).reshape(32)
  perm = jnp.concatenate([grp, grp + 32])
  W1a = W1[d:2 * d][:, perm]
  W1b = W1[2 * d:3 * d][:, perm]
  W1g = W1[3 * d:4 * d][:, perm]
  b1p = b1[perm]

  rows1 = 10000
  grid1 = n_atoms // rows1
  A1, A2 = pl.pallas_call(
      _atom_tables_body,
      grid=(grid1,),
      in_specs=[
          pl.BlockSpec((rows1, d), lambda i: (i, 0)),
          pl.BlockSpec((d, d), lambda i: (0, 0)),
          pl.BlockSpec((d, d), lambda i: (0, 0)),
      ],
      out_specs=[
          pl.BlockSpec((rows1, d), lambda i: (i, 0)),
          pl.BlockSpec((rows1, d), lambda i: (i, 0)),
      ],
      out_shape=[
          jax.ShapeDtypeStruct((n_atoms, d), jnp.bfloat16),
          jax.ShapeDtypeStruct((n_atoms, d), jnp.bfloat16),
      ],
  )(atom_feats, W1a, W1b)

  G1 = pl.pallas_call(
      _glob_table_body,
      out_shape=jax.ShapeDtypeStruct((global_feats.shape[0], d), jnp.bfloat16),
  )(global_feats, W1g, b1p.reshape(1, d))

  pad = _B_PAD - n_bonds
  i0 = jnp.pad(a2b_src[:, 0], (0, pad)).reshape(_NW, _N_CHUNKS, 1, _CHUNK)
  i1 = jnp.pad(a2b_src[:, 1], (0, pad)).reshape(_NW, _N_CHUNKS, 1, _CHUNK)
  ig = jnp.pad(g2b_src, (0, pad)).reshape(_NW, _N_CHUNKS, 1, _CHUNK)
  idx_all = jnp.concatenate([i0, i1, ig], axis=2).reshape(_NW, _IDX_PER_W)

  mesh = plsc.VectorSubcoreMesh(
      core_axis_name="c", subcore_axis_name="s",
      num_cores=_NC, num_subcores=_NS,
  )
  sc_gather = pl.kernel(
      _sc_gather_body,
      out_type=jax.ShapeDtypeStruct((_B_PAD // 2, 2 * d), jnp.float32),
      mesh=mesh,
      compiler_params=pltpu.CompilerParams(
          use_tc_tiling_on_sc=False, needs_layout_passes=False),
      scratch_types=[
          pltpu.VMEM((_IDX_PER_W,), jnp.int32),
          pltpu.VMEM((_CHUNK, d), jnp.bfloat16),
          pltpu.VMEM((_CHUNK, d), jnp.bfloat16),
          pltpu.VMEM((_CHUNK, d), jnp.bfloat16),
          pltpu.VMEM((_CHUNK, d), jnp.bfloat16),
          pltpu.VMEM((_CHUNK, d), jnp.bfloat16),
          pltpu.VMEM((_CHUNK, d), jnp.bfloat16),
          pltpu.VMEM((_CHUNK // 2, 2 * d), jnp.float32),
          pltpu.VMEM((_CHUNK // 2, 2 * d), jnp.float32),
          pltpu.SemaphoreType.DMA,
          pltpu.SemaphoreType.DMA,
          pltpu.SemaphoreType.DMA,
          pltpu.SemaphoreType.DMA,
      ],
  )
  S2 = sc_gather(A1, A2, G1, idx_all)

  eye2 = jnp.eye(2, dtype=jnp.float32)
  W1bd = jnp.kron(eye2, W1m)
  W2bd = jnp.kron(eye2, W2)
  W3bd = jnp.kron(eye2, W3)
  b2bd = jnp.tile(b2, 2).reshape(1, 2 * d)
  b3bd = jnp.tile(b3, 2).reshape(64, 1)
  master2 = master_feats.reshape(n_bonds // 2, 2 * d)

  rows3 = 2048
  grid3 = pl.cdiv(n_bonds // 2, rows3)
  out = pl.pallas_call(
      _mlp_body,
      grid=(grid3,),
      in_specs=[
          pl.BlockSpec((rows3, 2 * d), lambda i: (i, 0)),
          pl.BlockSpec((rows3, 2 * d), lambda i: (i, 0)),
          pl.BlockSpec((2 * d, 2 * d), lambda i: (0, 0)),
          pl.BlockSpec((2 * d, 2 * d), lambda i: (0, 0)),
          pl.BlockSpec((1, 2 * d), lambda i: (0, 0)),
          pl.BlockSpec((2 * d, 64), lambda i: (0, 0)),
          pl.BlockSpec((64, 1), lambda i: (0, 0)),
      ],
      out_specs=[
          pl.BlockSpec((32, rows3), lambda i: (0, i)),
          pl.BlockSpec((32, rows3), lambda i: (0, i)),
      ],
      out_shape=[
          jax.ShapeDtypeStruct((32, n_bonds // 2), jnp.float32),
          jax.ShapeDtypeStruct((32, n_bonds // 2), jnp.float32),
      ],
  )(master2, S2, W1bd, W2bd, b2bd, W3bd, b3bd)

  out_e, out_o = out
  out_t = jnp.stack([out_e, out_o], axis=2).reshape(32, n_bonds)
  return out_t.T


# speedup vs baseline: 1.1675x; 1.0005x over previous
import jax
import jax.numpy as jnp
from jax import lax
from jax.experimental import pallas as pl
from jax.experimental.pallas import tpu as pltpu
from jax.experimental.pallas import tpu_sc as plsc

_NC = 2
_NS = 16
_NW = _NC * _NS

_CHUNK = 128
_N_CHUNKS = 124
_PER_W = _CHUNK * _N_CHUNKS
_B_PAD = _NW * _PER_W

_D = 64
_ROWS_TC = 4000


def _softplus(x):
  return jnp.maximum(x, 0.0) + jnp.log1p(jnp.exp(-jnp.abs(x)))


def _atom_tables_body(a_ref, w1a_ref, w1b_ref, o1_ref, o2_ref):
  a = a_ref[...]
  o1_ref[...] = jnp.dot(
      a, w1a_ref[...], preferred_element_type=jnp.float32
  ).astype(jnp.bfloat16)
  o2_ref[...] = jnp.dot(
      a, w1b_ref[...], preferred_element_type=jnp.float32
  ).astype(jnp.bfloat16)


def _glob_table_body(g_ref, wg_ref, b1_ref, o_ref):
  o_ref[...] = (
      jnp.dot(g_ref[...], wg_ref[...], preferred_element_type=jnp.float32)
      + b1_ref[...]
  ).astype(jnp.bfloat16)


_IDX_PER_W = _N_CHUNKS * 3 * _CHUNK


def _sc_gather_body(a1_hbm, a2_hbm, g1_hbm, idx_hbm, out_hbm,
                    idx_v, bufa0, bufb0, bufg0, bufa1, bufb1, bufg1,
                    outb0, outb1, sem_g0, sem_g1, sem_o0, sem_o1):
  wid = lax.axis_index("s") * _NC + lax.axis_index("c")
  pbase0 = wid * (_PER_W // 2)

  pltpu.sync_copy(idx_hbm.at[wid], idx_v)

  sets = (
      (bufa0, bufb0, bufg0, outb0, sem_g0, sem_o0),
      (bufa1, bufb1, bufg1, outb1, sem_g1, sem_o1),
  )

  def issue(s, j):
    ba, bb, bg, _, sg, _ = sets[s]
    off = j * (3 * _CHUNK)
    pltpu.async_copy(a1_hbm.at[idx_v.at[pl.ds(off, _CHUNK)]], ba, sg)
    pltpu.async_copy(a2_hbm.at[idx_v.at[pl.ds(off + _CHUNK, _CHUNK)]], bb, sg)
    pltpu.async_copy(g1_hbm.at[idx_v.at[pl.ds(off + 2 * _CHUNK, _CHUNK)]],
                     bg, sg)

  def drain_gathers(s):
    ba, bb, bg, _, sg, _ = sets[s]
    pltpu.make_async_copy(a1_hbm.at[idx_v.at[pl.ds(0, _CHUNK)]], ba, sg).wait()
    pltpu.make_async_copy(a2_hbm.at[idx_v.at[pl.ds(0, _CHUNK)]], bb, sg).wait()
    pltpu.make_async_copy(g1_hbm.at[idx_v.at[pl.ds(0, _CHUNK)]], bg, sg).wait()

  def drain_out(s):
    _, _, _, ob, _, so = sets[s]
    pltpu.make_async_copy(ob, out_hbm.at[pl.ds(0, _CHUNK // 2)], so).wait()

  def consume(s, j):
    ba, bb, bg, ob, sg, so = sets[s]
    drain_gathers(s)

    def row_body(r, c2):
      for h in range(2):
        b = 2 * r + h
        for k in range(_D // 32):
          sl = pl.ds(k * 32, 32)
          ssum = ba[b, sl] + bb[b, sl] + bg[b, sl]
          lo, hi = plsc.unpack(ssum, format=plsc.PackFormat.INTERLEAVED)
          ob[r, pl.ds(h * _D + k * 32, 16)] = lo
          ob[r, pl.ds(h * _D + k * 32 + 16, 16)] = hi
      return c2

    lax.fori_loop(0, _CHUNK // 2, row_body, 0, unroll=4)
    pltpu.async_copy(
        ob, out_hbm.at[pl.ds(pbase0 + j * (_CHUNK // 2), _CHUNK // 2)], so)

  issue(0, 0)

  def pair_body(t, carry):
    j1 = 2 * t + 1
    j0n = 2 * t + 2

    @pl.when(t > 0)
    def _():
      drain_out(1)
    issue(1, j1)

    consume(0, 2 * t)

    @pl.when(j0n < _N_CHUNKS)
    def _():
      drain_out(0)
      issue(0, j0n)

    consume(1, j1)
    return carry

  lax.fori_loop(0, _N_CHUNKS // 2, pair_body, 0)
  drain_out(0)
  drain_out(1)


def _mlp_body(x_ref, s_ref, w1m_ref, w2_ref, b2_ref, w3_ref, b3_ref,
              oe_ref, oo_ref):
  h = jnp.dot(x_ref[...], w1m_ref[...], preferred_element_type=jnp.float32)
  h = _softplus(h + s_ref[...])
  h = _softplus(
      jnp.dot(h, w2_ref[...], preferred_element_type=jnp.float32) + b2_ref[...]
  )
  tpacked = lax.dot_general(
      w3_ref[...], h, (((0,), (1,)), ((), ())),
      preferred_element_type=jnp.float32,
  ) + b3_ref[...]
  oe_ref[...] = tpacked[:32, :]
  oo_ref[...] = tpacked[32:, :]


def kernel(master_feats, atom_feats, global_feats, a2b_src, g2b_src,
           W1, b1, W2, b2, W3, b3):
  n_bonds = master_feats.shape[0]
  n_atoms = atom_feats.shape[0]
  d = _D

  W1m = W1[:d]
  j16 = jnp.arange(16)
  grp = jnp.stack([j16, j16 + 16], axis=1).reshape(32)
  perm = jnp.concatenate([grp, grp + 32])
  W1a = W1[d:2 * d][:, perm]
  W1b = W1[2 * d:3 * d][:, perm]
  W1g = W1[3 * d:4 * d][:, perm]
  b1p = b1[perm]

  rows1 = 10000
  grid1 = n_atoms // rows1
  A1, A2 = pl.pallas_call(
      _atom_tables_body,
      grid=(grid1,),
      in_specs=[
          pl.BlockSpec((rows1, d), lambda i: (i, 0)),
          pl.BlockSpec((d, d), lambda i: (0, 0)),
          pl.BlockSpec((d, d), lambda i: (0, 0)),
      ],
      out_specs=[
          pl.BlockSpec((rows1, d), lambda i: (i, 0)),
          pl.BlockSpec((rows1, d), lambda i: (i, 0)),
      ],
      out_shape=[
          jax.ShapeDtypeStruct((n_atoms, d), jnp.bfloat16),
          jax.ShapeDtypeStruct((n_atoms, d), jnp.bfloat16),
      ],
  )(atom_feats, W1a, W1b)

  G1 = pl.pallas_call(
      _glob_table_body,
      out_shape=jax.ShapeDtypeStruct((global_feats.shape[0], d), jnp.bfloat16),
  )(global_feats, W1g, b1p.reshape(1, d))

  pad = _B_PAD - n_bonds
  i0 = jnp.pad(a2b_src[:, 0], (0, pad)).reshape(_NW, _N_CHUNKS, 1, _CHUNK)
  i1 = jnp.pad(a2b_src[:, 1], (0, pad)).reshape(_NW, _N_CHUNKS, 1, _CHUNK)
  ig = jnp.pad(g2b_src, (0, pad)).reshape(_NW, _N_CHUNKS, 1, _CHUNK)
  idx_all = jnp.concatenate([i0, i1, ig], axis=2).reshape(_NW, _IDX_PER_W)

  mesh = plsc.VectorSubcoreMesh(
      core_axis_name="c", subcore_axis_name="s",
      num_cores=_NC, num_subcores=_NS,
  )
  sc_gather = pl.kernel(
      _sc_gather_body,
      out_type=jax.ShapeDtypeStruct((_B_PAD // 2, 2 * d), jnp.float32),
      mesh=mesh,
      compiler_params=pltpu.CompilerParams(
          use_tc_tiling_on_sc=False, needs_layout_passes=False),
      scratch_types=[
          pltpu.VMEM((_IDX_PER_W,), jnp.int32),
          pltpu.VMEM((_CHUNK, d), jnp.bfloat16),
          pltpu.VMEM((_CHUNK, d), jnp.bfloat16),
          pltpu.VMEM((_CHUNK, d), jnp.bfloat16),
          pltpu.VMEM((_CHUNK, d), jnp.bfloat16),
          pltpu.VMEM((_CHUNK, d), jnp.bfloat16),
          pltpu.VMEM((_CHUNK, d), jnp.bfloat16),
          pltpu.VMEM((_CHUNK // 2, 2 * d), jnp.float32),
          pltpu.VMEM((_CHUNK // 2, 2 * d), jnp.float32),
          pltpu.SemaphoreType.DMA,
          pltpu.SemaphoreType.DMA,
          pltpu.SemaphoreType.DMA,
          pltpu.SemaphoreType.DMA,
      ],
  )
  S2 = sc_gather(A1, A2, G1, idx_all)

  eye2 = jnp.eye(2, dtype=jnp.float32)
  W1bd = jnp.kron(eye2, W1m)
  W2bd = jnp.kron(eye2, W2)
  W3bd = jnp.kron(eye2, W3)
  b2bd = jnp.tile(b2, 2).reshape(1, 2 * d)
  b3bd = jnp.tile(b3, 2).reshape(64, 1)
  master2 = master_feats.reshape(n_bonds // 2, 2 * d)

  rows3 = 2048
  grid3 = pl.cdiv(n_bonds // 2, rows3)
  out = pl.pallas_call(
      _mlp_body,
      grid=(grid3,),
      in_specs=[
          pl.BlockSpec((rows3, 2 * d), lambda i: (i, 0)),
          pl.BlockSpec((rows3, 2 * d), lambda i: (i, 0)),
          pl.BlockSpec((2 * d, 2 * d), lambda i: (0, 0)),
          pl.BlockSpec((2 * d, 2 * d), lambda i: (0, 0)),
          pl.BlockSpec((1, 2 * d), lambda i: (0, 0)),
          pl.BlockSpec((2 * d, 64), lambda i: (0, 0)),
          pl.BlockSpec((64, 1), lambda i: (0, 0)),
      ],
      out_specs=[
          pl.BlockSpec((32, rows3), lambda i: (0, i)),
          pl.BlockSpec((32, rows3), lambda i: (0, i)),
      ],
      out_shape=[
          jax.ShapeDtypeStruct((32, n_bonds // 2), jnp.float32),
          jax.ShapeDtypeStruct((32, n_bonds // 2), jnp.float32),
      ],
  )(master2, S2, W1bd, W2bd, b2bd, W3bd, b3bd)

  out_e, out_o = out
  out_t = jnp.stack([out_e, out_o], axis=2).reshape(32, n_bonds)
  return out_t.T
